# Initial kernel scaffold; baseline (speedup 1.0000x reference)
import jax
import jax.numpy as jnp
from jax.experimental import pallas as pl


def kernel(x, resp_table, act_table, W, b):
    raise NotImplementedError("write your pallas kernel here")


# speedup vs baseline: 10.2494x; 10.2494x over previous
import functools

import jax
import jax.numpy as jnp
from jax import lax
from jax.experimental import pallas as pl
from jax.experimental.pallas import tpu as pltpu
from jax.experimental.pallas import tpu_sc as plsc

B = 4096
L = 200
N_TOK = B * L
EDIM = 16
TOK_DIM = 64

NUM_WORKERS = 32
PER_W = N_TOK // NUM_WORKERS
TBATCH = 128
K_INFLIGHT = 20
GROUP = TBATCH * K_INFLIGHT
NUM_GROUPS = PER_W // GROUP


_sc_mesh = plsc.VectorSubcoreMesh(core_axis_name="c", subcore_axis_name="s")


@functools.partial(
    pl.kernel,
    mesh=_sc_mesh,
    out_type=[
        jax.ShapeDtypeStruct((N_TOK, EDIM), jnp.float32),
        jax.ShapeDtypeStruct((N_TOK, EDIM), jnp.float32),
    ],
    scratch_types=[
        pltpu.VMEM((GROUP,), jnp.int32),
        pltpu.VMEM((GROUP,), jnp.int32),
        pltpu.VMEM((GROUP, EDIM), jnp.float32),
        pltpu.VMEM((GROUP, EDIM), jnp.float32),
        pltpu.SemaphoreType.DMA,
        pltpu.SemaphoreType.DMA,
    ],
    compiler_params=pltpu.CompilerParams(use_tc_tiling_on_sc=False),
)
def _sc_gather(ridx_hbm, aidx_hbm, rtab_hbm, atab_hbm, out_r_hbm, out_a_hbm,
               idxr_v, idxa_v, bufr_v, bufa_v, semr, sema):
    wid = lax.axis_index("s") * 2 + lax.axis_index("c")
    base = wid * PER_W

    def group_body(g, carry):
        goff = base + g * GROUP
        pltpu.sync_copy(ridx_hbm.at[pl.ds(goff, GROUP)], idxr_v)
        pltpu.sync_copy(aidx_hbm.at[pl.ds(goff, GROUP)], idxa_v)
        rcopies = []
        acopies = []
        for t in range(K_INFLIGHT):
            sl = pl.ds(t * TBATCH, TBATCH)
            rcopies.append(
                pltpu.async_copy(rtab_hbm.at[idxr_v.at[sl]], bufr_v.at[sl], semr))
            acopies.append(
                pltpu.async_copy(atab_hbm.at[idxa_v.at[sl]], bufa_v.at[sl], sema))
        for c in rcopies:
            c.wait()
        pltpu.sync_copy(bufr_v, out_r_hbm.at[pl.ds(goff, GROUP)])
        for c in acopies:
            c.wait()
        pltpu.sync_copy(bufa_v, out_a_hbm.at[pl.ds(goff, GROUP)])
        return carry

    lax.fori_loop(0, NUM_GROUPS, group_body, 0)


BLK = 4096


def _proj_body(gr_ref, ga_ref, ct_ref, wr_ref, wa_ref, wc_ref, b_ref, o_ref):
    acc = jnp.dot(gr_ref[...], wr_ref[...], preferred_element_type=jnp.float32)
    acc = acc + jnp.dot(ga_ref[...], wa_ref[...], preferred_element_type=jnp.float32)
    acc = acc + jnp.dot(ct_ref[...], wc_ref[...], preferred_element_type=jnp.float32)
    o_ref[...] = acc + b_ref[...]


def _tc_project(gr, ga, ct, wr, wa, wc, b2):
    grid = N_TOK // BLK
    return pl.pallas_call(
        _proj_body,
        grid=(grid,),
        in_specs=[
            pl.BlockSpec((BLK, EDIM), lambda i: (i, 0)),
            pl.BlockSpec((BLK, EDIM), lambda i: (i, 0)),
            pl.BlockSpec((BLK, 2), lambda i: (i, 0)),
            pl.BlockSpec((EDIM, TOK_DIM), lambda i: (0, 0)),
            pl.BlockSpec((EDIM, TOK_DIM), lambda i: (0, 0)),
            pl.BlockSpec((2, TOK_DIM), lambda i: (0, 0)),
            pl.BlockSpec((1, TOK_DIM), lambda i: (0, 0)),
        ],
        out_specs=pl.BlockSpec((BLK, TOK_DIM), lambda i: (i, 0)),
        out_shape=jax.ShapeDtypeStruct((N_TOK, TOK_DIM), jnp.float32),
        compiler_params=pltpu.CompilerParams(
            dimension_semantics=("arbitrary",),
        ),
    )(gr, ga, ct, wr, wa, wc, b2)


def kernel(x, resp_table, act_table, W, b):
    xf = x.reshape(N_TOK, 4)
    ridx = xf[:, 0].astype(jnp.int32)
    aidx = xf[:, 1].astype(jnp.int32)
    ct = xf[:, 2:4]
    gr, ga = _sc_gather(ridx, aidx, resp_table, act_table)
    out = _tc_project(gr, ga, ct, W[0:16], W[16:32], W[32:34],
                      b.reshape(1, TOK_DIM))
    return out.reshape(B, L, TOK_DIM)


# speedup vs baseline: 19.1401x; 1.8674x over previous
import functools

import jax
import jax.numpy as jnp
from jax import lax
from jax.experimental import pallas as pl
from jax.experimental.pallas import tpu as pltpu
from jax.experimental.pallas import tpu_sc as plsc

B = 4096
L = 200
N_TOK = B * L
EDIM = 16
TOK_DIM = 64

NUM_WORKERS = 32
PER_W = N_TOK // NUM_WORKERS
TBATCH = 128
K_INFLIGHT = 20
GROUP = TBATCH * K_INFLIGHT
NUM_GROUPS = PER_W // GROUP


@functools.cache
def _make_sc_gather():
    mesh = plsc.VectorSubcoreMesh(core_axis_name="c", subcore_axis_name="s")

    @functools.partial(
        pl.kernel,
        mesh=mesh,
        out_type=[
            jax.ShapeDtypeStruct((N_TOK, EDIM), jnp.float32),
            jax.ShapeDtypeStruct((N_TOK, EDIM), jnp.float32),
        ],
        scratch_types=[
            pltpu.VMEM((GROUP,), jnp.int32),
            pltpu.VMEM((GROUP,), jnp.int32),
            pltpu.VMEM((GROUP, EDIM), jnp.float32),
            pltpu.VMEM((GROUP, EDIM), jnp.float32),
            pltpu.SemaphoreType.DMA,
            pltpu.SemaphoreType.DMA,
        ],
        compiler_params=pltpu.CompilerParams(use_tc_tiling_on_sc=False),
    )
    def _sc_gather(ridx_hbm, aidx_hbm, rtab_hbm, atab_hbm, out_r_hbm, out_a_hbm,
                   idxr_v, idxa_v, bufr_v, bufa_v, semr, sema):
        wid = lax.axis_index("s") * 2 + lax.axis_index("c")
        base = wid * PER_W

        def group_body(g, carry):
            goff = base + g * GROUP
            pltpu.sync_copy(ridx_hbm.at[pl.ds(goff, GROUP)], idxr_v)
            pltpu.sync_copy(aidx_hbm.at[pl.ds(goff, GROUP)], idxa_v)
            rcopies = []
            acopies = []
            for t in range(K_INFLIGHT):
                sl = pl.ds(t * TBATCH, TBATCH)
                rcopies.append(
                    pltpu.async_copy(rtab_hbm.at[idxr_v.at[sl]], bufr_v.at[sl],
                                     semr))
                acopies.append(
                    pltpu.async_copy(atab_hbm.at[idxa_v.at[sl]], bufa_v.at[sl],
                                     sema))
            for c in rcopies:
                c.wait()
            pltpu.sync_copy(bufr_v, out_r_hbm.at[pl.ds(goff, GROUP)])
            for c in acopies:
                c.wait()
            pltpu.sync_copy(bufa_v, out_a_hbm.at[pl.ds(goff, GROUP)])
            return carry

        lax.fori_loop(0, NUM_GROUPS, group_body, 0)

    return _sc_gather


def _proj_body(gr_ref, ga_ref, ct_ref, wr_ref, wa_ref, wct_ref, bc_ref,
               o_ref):
    cdims = (((0,), (1,)), ((), ()))
    acc = lax.dot_general(wr_ref[...], gr_ref[...], cdims,
                          preferred_element_type=jnp.float32)
    acc = acc + lax.dot_general(wa_ref[...], ga_ref[...], cdims,
                                preferred_element_type=jnp.float32)
    acc = acc + lax.dot_general(wct_ref[...], ct_ref[0],
                                (((1,), (0,)), ((), ())),
                                preferred_element_type=jnp.float32)
    o_ref[...] = (acc + bc_ref[...])[None]


def _tc_project(gr, ga, ct, wr, wa, wct, bc):
    return pl.pallas_call(
        _proj_body,
        grid=(L,),
        in_specs=[
            pl.BlockSpec((B, EDIM), lambda i: (i, 0)),
            pl.BlockSpec((B, EDIM), lambda i: (i, 0)),
            pl.BlockSpec((1, 2, B), lambda i: (i, 0, 0)),
            pl.BlockSpec((EDIM, TOK_DIM), lambda i: (0, 0)),
            pl.BlockSpec((EDIM, TOK_DIM), lambda i: (0, 0)),
            pl.BlockSpec((TOK_DIM, 2), lambda i: (0, 0)),
            pl.BlockSpec((TOK_DIM, 1), lambda i: (0, 0)),
        ],
        out_specs=pl.BlockSpec((1, TOK_DIM, B), lambda i: (i, 0, 0)),
        out_shape=jax.ShapeDtypeStruct((L, TOK_DIM, B), jnp.float32),
        compiler_params=pltpu.CompilerParams(
            dimension_semantics=("arbitrary",),
        ),
    )(gr, ga, ct, wr, wa, wct, bc)


def kernel(x, resp_table, act_table, W, b):
    ridx = x[:, :, 0].T.reshape(N_TOK).astype(jnp.int32)
    aidx = x[:, :, 1].T.reshape(N_TOK).astype(jnp.int32)
    ct = jnp.stack([x[:, :, 2].T, x[:, :, 3].T], axis=1)
    gr, ga = _make_sc_gather()(ridx, aidx, resp_table, act_table)
    out3 = _tc_project(gr, ga, ct, W[0:16], W[16:32], W[32:34].T,
                       b.reshape(TOK_DIM, 1))
    return jnp.transpose(out3, (2, 0, 1))


# speedup vs baseline: 20.7698x; 1.0852x over previous
import functools

import jax
import jax.numpy as jnp
from jax import lax
from jax.experimental import pallas as pl
from jax.experimental.pallas import tpu as pltpu
from jax.experimental.pallas import tpu_sc as plsc

B = 4096
L = 200
N_TOK = B * L
EDIM = 16
TOK_DIM = 64
VOCAB = 1000000
GROWS = B * EDIM // 128

NUM_WORKERS = 32
PER_W = N_TOK // NUM_WORKERS
TBATCH = 128
K_INFLIGHT = 20
GROUP = TBATCH * K_INFLIGHT
NUM_GROUPS = PER_W // GROUP


@functools.cache
def _make_sc_gather():
    mesh = plsc.VectorSubcoreMesh(core_axis_name="c", subcore_axis_name="s")

    @functools.partial(
        pl.kernel,
        mesh=mesh,
        out_type=[
            jax.ShapeDtypeStruct((N_TOK, EDIM), jnp.float32),
            jax.ShapeDtypeStruct((N_TOK, EDIM), jnp.float32),
        ],
        scratch_types=[
            pltpu.VMEM((GROUP,), jnp.int32),
            pltpu.VMEM((GROUP,), jnp.int32),
            pltpu.VMEM((GROUP, EDIM), jnp.float32),
            pltpu.VMEM((GROUP, EDIM), jnp.float32),
            pltpu.SemaphoreType.DMA,
            pltpu.SemaphoreType.DMA,
        ],
        compiler_params=pltpu.CompilerParams(use_tc_tiling_on_sc=False),
    )
    def _sc_gather(ridx_hbm, aidx_hbm, rtab_hbm, atab_hbm, out_r_hbm,
                   out_a_hbm, idxr_v, idxa_v, bufr_v, bufa_v, semr, sema):
        wid = lax.axis_index("s") * 2 + lax.axis_index("c")
        base = wid * PER_W

        def group_body(g, carry):
            goff = base + g * GROUP
            pltpu.sync_copy(ridx_hbm.at[pl.ds(goff, GROUP)], idxr_v)
            pltpu.sync_copy(aidx_hbm.at[pl.ds(goff, GROUP)], idxa_v)
            rcopies = []
            acopies = []
            for t in range(K_INFLIGHT):
                sl = pl.ds(t * TBATCH, TBATCH)
                rcopies.append(
                    pltpu.async_copy(rtab_hbm.at[idxr_v.at[sl]], bufr_v.at[sl],
                                     semr))
                acopies.append(
                    pltpu.async_copy(atab_hbm.at[idxa_v.at[sl]], bufa_v.at[sl],
                                     sema))
            for c in rcopies:
                c.wait()
            pltpu.sync_copy(bufr_v, out_r_hbm.at[pl.ds(goff, GROUP)])
            for c in acopies:
                c.wait()
            pltpu.sync_copy(bufa_v, out_a_hbm.at[pl.ds(goff, GROUP)])
            return carry

        lax.fori_loop(0, NUM_GROUPS, group_body, 0)

    return _sc_gather


def _proj_body(gr_ref, ga_ref, ct_ref, wr_ref, wa_ref, wc_ref, bb_ref,
               o_ref):
    acc = jnp.dot(gr_ref[...], wr_ref[...], preferred_element_type=jnp.float32)
    acc = acc + jnp.dot(ga_ref[...], wa_ref[...],
                        preferred_element_type=jnp.float32)
    acc = acc + jnp.dot(ct_ref[...], wc_ref[...],
                        preferred_element_type=jnp.float32)
    o_ref[...] = acc + bb_ref[...]


def _tc_project(gr128, ga128, ct512, wbd_r, wbd_a, wbd_c, bb):
    return pl.pallas_call(
        _proj_body,
        grid=(L,),
        in_specs=[
            pl.BlockSpec((GROWS, 128), lambda i: (i, 0)),
            pl.BlockSpec((GROWS, 128), lambda i: (i, 0)),
            pl.BlockSpec((GROWS, EDIM), lambda i: (i, 0)),
            pl.BlockSpec((128, 8 * TOK_DIM), lambda i: (0, 0)),
            pl.BlockSpec((128, 8 * TOK_DIM), lambda i: (0, 0)),
            pl.BlockSpec((EDIM, 8 * TOK_DIM), lambda i: (0, 0)),
            pl.BlockSpec((1, 8 * TOK_DIM), lambda i: (0, 0)),
        ],
        out_specs=pl.BlockSpec((GROWS, 8 * TOK_DIM), lambda i: (i, 0)),
        out_shape=jax.ShapeDtypeStruct((L * GROWS, 8 * TOK_DIM), jnp.float32),
        compiler_params=pltpu.CompilerParams(
            dimension_semantics=("arbitrary",),
        ),
    )(gr128, ga128, ct512, wbd_r, wbd_a, wbd_c, bb)


def kernel(x, resp_table, act_table, W, b):
    ridx = x[:, :, 0].T.reshape(N_TOK).astype(jnp.int32)
    aidx = x[:, :, 1].T.reshape(N_TOK).astype(jnp.int32)
    c2 = x[:, :, 2].T.reshape(L, GROWS, 8)
    c3 = x[:, :, 3].T.reshape(L, GROWS, 8)
    ct512 = jnp.stack([c2, c3], axis=-1).reshape(L * GROWS, EDIM)
    gr, ga = _make_sc_gather()(ridx, aidx, resp_table, act_table)
    gr128 = gr.reshape(L * GROWS, 128)
    ga128 = ga.reshape(L * GROWS, 128)
    eye8 = jnp.eye(8, dtype=jnp.float32)
    wbd_r = jnp.kron(eye8, W[0:16])
    wbd_a = jnp.kron(eye8, W[16:32])
    wbd_c = jnp.kron(eye8, W[32:34])
    bb = jnp.tile(b, 8).reshape(1, 8 * TOK_DIM)
    out128 = _tc_project(gr128, ga128, ct512, wbd_r, wbd_a, wbd_c, bb)
    return jnp.transpose(out128.reshape(L, B, TOK_DIM), (1, 0, 2))
